# baseline (speedup 1.0000x reference)
import functools

import jax
import jax.numpy as jnp
from jax import lax
from jax.experimental import pallas as pl
from jax.experimental.pallas import tpu as pltpu
from jax.experimental.pallas import tpu_sc as plsc

N = 10000
NP = 10240
E = 160000
EPAD = 163840
EP = 100000
EPP = 102400
H = 256
HH = 128
MLPD = 512
EF = 75
AUGP = 640

NC = 2
NS = 16
NW = NC * NS

_f32 = jnp.float32
_i32 = jnp.int32


def _sc_mesh():
    return plsc.VectorSubcoreMesh(core_axis_name="c", subcore_axis_name="s")


def _emb_gather(x2d, emb):
    nrows = NP // 64
    per_w = nrows // NW

    def body(x_ref, emb_ref, out_ref, idx_v, rows_v,
             sem_g0, sem_g1, sem_w0, sem_w1):
        c = lax.axis_index("c")
        s = lax.axis_index("s")
        w = s * NC + c
        sem_g = (sem_g0, sem_g1)
        sem_w = (sem_w0, sem_w1)
        pltpu.sync_copy(x_ref.at[w], idx_v)
        gathers = [None] * per_w
        writes = [None] * per_w
        for j in range(per_w):
            b = j % 2
            if j >= 2:
                writes[j - 2].wait()
            gathers[j] = pltpu.async_copy(
                emb_ref.at[idx_v.at[j]], rows_v.at[b], sem_g[b])
            if j >= 1:
                p = j - 1
                gathers[p].wait()
                base = pl.multiple_of((w * per_w + p) * 64, 64)
                writes[p] = pltpu.async_copy(
                    rows_v.at[p % 2], out_ref.at[pl.ds(base, 64)],
                    sem_w[p % 2])
        p = per_w - 1
        gathers[p].wait()
        base = pl.multiple_of((w * per_w + p) * 64, 64)
        writes[p] = pltpu.async_copy(
            rows_v.at[p % 2], out_ref.at[pl.ds(base, 64)], sem_w[p % 2])
        writes[per_w - 2].wait()
        writes[per_w - 1].wait()

    k = pl.kernel(
        body,
        out_type=jax.ShapeDtypeStruct((NP, H), _f32),
        mesh=_sc_mesh(),
        scratch_types=[
            pltpu.VMEM((per_w, 64), _i32),
            pltpu.VMEM((2, 64, H), _f32),
            pltpu.SemaphoreType.DMA,
            pltpu.SemaphoreType.DMA,
            pltpu.SemaphoreType.DMA,
            pltpu.SemaphoreType.DMA,
        ],
    )
    return k(x2d, emb)


def _segsum(hf, srcp, dstp, zfeat):
    rows_per_tile = NP // NS
    chunks = EPAD // (NS * 128)
    SUP = 8

    def body(h_ref, src_ref, dst_ref, zf_ref, neigh_ref,
             src_v, dst_v, gidx_v, rows_v, acc,
             sem_g0, sem_g1, sem_s0, sem_s1):
        c = lax.axis_index("c")
        s = lax.axis_index("s")
        stripe = pl.ds(pl.multiple_of(s * rows_per_tile, rows_per_tile),
                       rows_per_tile)
        sem_g = (sem_g0, sem_g1)
        sem_s = (sem_s0, sem_s1)
        pltpu.sync_copy(zf_ref, acc.at[stripe])
        plsc.subcore_barrier()

        gathers = [None] * chunks
        scats = [None] * chunks
        for t in range(chunks):
            b = t % 2
            sup, j = divmod(t, SUP)
            sb = sup % 2
            if j == 0:
                koff = pl.multiple_of(sup * SUP, SUP)
                pltpu.sync_copy(src_ref.at[s, pl.ds(koff, SUP)],
                                src_v.at[sb])
                pltpu.sync_copy(dst_ref.at[s, pl.ds(koff, SUP)],
                                dst_v.at[sb])
            if t >= 2:
                scats[t - 2].wait()
            for i in range(8):
                sl = pl.ds(i * 16, 16)
                gidx_v[b, sl] = src_v[sb, j, sl] * 2 + c
            gathers[t] = pltpu.async_copy(
                h_ref.at[gidx_v.at[b]], rows_v.at[b], sem_g[b])
            if t >= 1:
                p = t - 1
                gathers[p].wait()
                psup, pj = divmod(p, SUP)
                scats[p] = pltpu.async_copy(
                    rows_v.at[p % 2], acc.at[dst_v.at[psup % 2, pj]],
                    sem_s[p % 2], add=True)
        p = chunks - 1
        gathers[p].wait()
        psup, pj = divmod(p, SUP)
        scats[p] = pltpu.async_copy(
            rows_v.at[p % 2], acc.at[dst_v.at[psup % 2, pj]],
            sem_s[p % 2], add=True)
        scats[chunks - 2].wait()
        scats[chunks - 1].wait()
        plsc.subcore_barrier()
        pltpu.sync_copy(acc.at[stripe], neigh_ref.at[c, stripe])

    k = pl.kernel(
        body,
        out_type=jax.ShapeDtypeStruct((2, NP, 128), _f32),
        mesh=_sc_mesh(),
        scratch_types=[
            pltpu.VMEM((2, SUP, 128), _i32),
            pltpu.VMEM((2, SUP, 128), _i32),
            pltpu.VMEM((2, 128), _i32),
            pltpu.VMEM((2, 128, 128), _f32),
            pltpu.VMEM_SHARED((NP, 128), _f32),
            pltpu.SemaphoreType.DMA,
            pltpu.SemaphoreType.DMA,
            pltpu.SemaphoreType.DMA,
            pltpu.SemaphoreType.DMA,
        ],
    )
    return k(hf, srcp, dstp, zfeat)


def _deg(dstw, ones_hbm, zfeat):
    rows_per_tile = NP // NS
    chunks = dstw.shape[1]

    def body(dst_ref, ones_ref, zf_ref, deg_ref, dst_v, ones_v, acc):
        c = lax.axis_index("c")
        s = lax.axis_index("s")
        w = s * NC + c
        stripe = pl.ds(pl.multiple_of(s * rows_per_tile, rows_per_tile),
                       rows_per_tile)
        pltpu.sync_copy(zf_ref, acc.at[stripe])
        pltpu.sync_copy(ones_ref, ones_v)
        pltpu.sync_copy(dst_ref.at[w], dst_v)
        plsc.subcore_barrier()

        def chunk(j, _):
            pltpu.sync_copy(ones_v, acc.at[dst_v.at[j]], add=True)
            return 0

        lax.fori_loop(0, chunks, chunk, 0)
        plsc.subcore_barrier()
        pltpu.sync_copy(acc.at[stripe], deg_ref.at[c, stripe])

    k = pl.kernel(
        body,
        out_type=jax.ShapeDtypeStruct((2, NP, 128), _f32),
        mesh=_sc_mesh(),
        scratch_types=[
            pltpu.VMEM((chunks, 128), _i32),
            pltpu.VMEM((128, 128), _f32),
            pltpu.VMEM_SHARED((NP, 128), _f32),
        ],
    )
    return k(dstw, ones_hbm, zfeat)


def _pair_gather(h2, psp, pdp):
    chunks = psp.shape[1]

    def body(h_ref, ps_ref, pd_ref, hs_ref, hd_ref,
             is_v, id_v, rows_v, sem_g0, sem_g1, sem_w0, sem_w1):
        c = lax.axis_index("c")
        s = lax.axis_index("s")
        w = s * NC + c
        sem_g = (sem_g0, sem_g1)
        sem_w = (sem_w0, sem_w1)
        pltpu.sync_copy(ps_ref.at[w], is_v)
        pltpu.sync_copy(pd_ref.at[w], id_v)

        n = 2 * chunks
        gathers = [None] * n
        writes = [None] * n

        def issue_write(u):
            j, which = divmod(u, 2)
            base = pl.multiple_of((w * chunks + j) * 128, 128)
            out = hs_ref if which == 0 else hd_ref
            writes[u] = pltpu.async_copy(
                rows_v.at[u % 2], out.at[pl.ds(base, 128)], sem_w[u % 2])

        for u in range(n):
            b = u % 2
            j, which = divmod(u, 2)
            if u >= 2:
                writes[u - 2].wait()
            idx = is_v if which == 0 else id_v
            gathers[u] = pltpu.async_copy(
                h_ref.at[idx.at[j]], rows_v.at[b], sem_g[b])
            if u >= 1:
                gathers[u - 1].wait()
                issue_write(u - 1)
        gathers[n - 1].wait()
        issue_write(n - 1)
        writes[n - 2].wait()
        writes[n - 1].wait()

    rows = NW * chunks * 128
    k = pl.kernel(
        body,
        out_type=(jax.ShapeDtypeStruct((rows, H), _f32),
                  jax.ShapeDtypeStruct((rows, H), _f32)),
        mesh=_sc_mesh(),
        scratch_types=[
            pltpu.VMEM((chunks, 128), _i32),
            pltpu.VMEM((chunks, 128), _i32),
            pltpu.VMEM((2, 128, H), _f32),
            pltpu.SemaphoreType.DMA,
            pltpu.SemaphoreType.DMA,
            pltpu.SemaphoreType.DMA,
            pltpu.SemaphoreType.DMA,
        ],
    )
    return k(h2, psp, pdp)


def _sage_combine(h, neigh, deg, Ws, Wn, b, mask_pad, out_bf16=False):
    BS = 1024
    grid = NP // BS
    odt = jnp.bfloat16 if out_bf16 else _f32

    def body(h_ref, n_ref, d_ref, ws_ref, wn_ref, b_ref, out_ref):
        g = pl.program_id(0)
        deg_col = d_ref[0][:, 0:1] + d_ref[1][:, 0:1]
        r = 1.0 / jnp.clip(deg_col, 1.0, None)
        nm = sum(
            jax.lax.dot_general(n_ref[q], wn_ref[q * 128:(q + 1) * 128, :],
                                (((1,), (0,)), ((), ())),
                                preferred_element_type=_f32)
            for q in range(2))
        out = (jnp.dot(h_ref[...], ws_ref[...], preferred_element_type=_f32)
               + nm * r + b_ref[...])
        if mask_pad:
            rows = g * BS + lax.broadcasted_iota(_i32, (BS, H), 0)
            out = jnp.where(rows < N, out, 0.0)
        out_ref[...] = out.astype(odt)

    return pl.pallas_call(
        body,
        grid=(grid,),
        in_specs=[
            pl.BlockSpec((BS, H), lambda g: (g, 0)),
            pl.BlockSpec((2, BS, 128), lambda g: (0, g, 0)),
            pl.BlockSpec((2, BS, 128), lambda g: (0, g, 0)),
            pl.BlockSpec((H, H), lambda g: (0, 0)),
            pl.BlockSpec((H, H), lambda g: (0, 0)),
            pl.BlockSpec((1, H), lambda g: (0, 0)),
        ],
        out_specs=pl.BlockSpec((BS, H), lambda g: (g, 0)),
        out_shape=jax.ShapeDtypeStruct((NP, H), odt),
    )(h, neigh, deg, Ws, Wn, b)


def _ef_stats(efp):
    BS = 1600
    grid = EPP // BS

    def body(ef_ref, out_ref):
        g = pl.program_id(0)
        s1 = jnp.sum(ef_ref[...], axis=0)[None, :]
        s2 = jnp.sum(ef_ref[...] * ef_ref[...], axis=0)[None, :]
        upd = jnp.concatenate([s1, s2, jnp.zeros((6, HH), _f32)], axis=0)

        @pl.when(g == 0)
        def _():
            out_ref[...] = upd

        @pl.when(g > 0)
        def _():
            out_ref[...] = out_ref[...] + upd

    return pl.pallas_call(
        body,
        grid=(grid,),
        in_specs=[pl.BlockSpec((BS, HH), lambda g: (g, 0))],
        out_specs=pl.BlockSpec((8, HH), lambda g: (0, 0)),
        out_shape=jax.ShapeDtypeStruct((8, HH), _f32),
        compiler_params=pltpu.CompilerParams(
            dimension_semantics=("arbitrary",)),
    )(efp)


def _h2_stats(h2, cs, cd):
    BS = 1024
    grid = NP // BS

    def body(h_ref, cs_ref, cd_ref, out_ref):
        g = pl.program_id(0)
        h = h_ref[...]
        hsq = h * h
        cs_col = cs_ref[0][:, 0:1] + cs_ref[1][:, 0:1]
        cd_col = cd_ref[0][:, 0:1] + cd_ref[1][:, 0:1]
        upd = jnp.concatenate([
            jnp.sum(h * cs_col, axis=0)[None, :],
            jnp.sum(hsq * cs_col, axis=0)[None, :],
            jnp.sum(h * cd_col, axis=0)[None, :],
            jnp.sum(hsq * cd_col, axis=0)[None, :],
            jnp.zeros((4, H), _f32),
        ], axis=0)

        @pl.when(g == 0)
        def _():
            out_ref[...] = upd

        @pl.when(g > 0)
        def _():
            out_ref[...] = out_ref[...] + upd

    return pl.pallas_call(
        body,
        grid=(grid,),
        in_specs=[
            pl.BlockSpec((BS, H), lambda g: (g, 0)),
            pl.BlockSpec((2, BS, 128), lambda g: (0, g, 0)),
            pl.BlockSpec((2, BS, 128), lambda g: (0, g, 0)),
        ],
        out_specs=pl.BlockSpec((8, H), lambda g: (0, 0)),
        out_shape=jax.ShapeDtypeStruct((8, H), _f32),
        compiler_params=pltpu.CompilerParams(
            dimension_semantics=("arbitrary",)),
    )(h2, cs, cd)


def _mlp(hs, hd, efp, ef_row0, scale, shift, W1p, b1, W2, b2, W3, b3, W4,
         b4, W5, b5, W6, b6):
    BS = 1024
    rows = hs.shape[0]
    grid = rows // BS
    ef_blk0 = ef_row0 // BS

    def body(hs_ref, hd_ref, ef_ref, sc_ref, sh_ref, w1_ref, b1_ref,
             w2_ref, b2_ref, w3_ref, b3_ref, w4_ref, b4_ref, w5_ref, b5_ref,
             w6_ref, b6_ref, out_ref):
        x = jnp.concatenate([hs_ref[...], hd_ref[...], ef_ref[...]], axis=1)
        a = x * sc_ref[...] + sh_ref[...]
        for w_ref, bb_ref in ((w1_ref, b1_ref), (w2_ref, b2_ref),
                              (w3_ref, b3_ref), (w4_ref, b4_ref),
                              (w5_ref, b5_ref)):
            a = jnp.maximum(
                jnp.dot(a, w_ref[...], preferred_element_type=_f32)
                + bb_ref[...], 0.0)
        logits = jnp.dot(a, w6_ref[...], preferred_element_type=_f32) \
            + b6_ref[...]
        l0 = logits[:, 0:1]
        l1 = logits[:, 1:2]
        m = jnp.maximum(l0, l1)
        e0 = jnp.exp(l0 - m)
        e1 = jnp.exp(l1 - m)
        inv = 1.0 / (e0 + e1)
        out_ref[...] = jnp.concatenate([e0 * inv, e1 * inv], axis=1)

    def wspec(shape):
        return pl.BlockSpec(shape, lambda g, _n=len(shape): (0,) * _n)

    return pl.pallas_call(
        body,
        grid=(grid,),
        in_specs=[
            pl.BlockSpec((BS, H), lambda g: (g, 0)),
            pl.BlockSpec((BS, H), lambda g: (g, 0)),
            pl.BlockSpec((BS, HH), lambda g: (ef_blk0 + g, 0)),
            wspec((1, AUGP)), wspec((1, AUGP)),
            wspec((AUGP, MLPD)), wspec((1, MLPD)),
            wspec((MLPD, MLPD)), wspec((1, MLPD)),
            wspec((MLPD, MLPD)), wspec((1, MLPD)),
            wspec((MLPD, MLPD)), wspec((1, MLPD)),
            wspec((MLPD, MLPD)), wspec((1, MLPD)),
            wspec((MLPD, 2)), wspec((1, 2)),
        ],
        out_specs=pl.BlockSpec((BS, 2), lambda g: (g, 0)),
        out_shape=jax.ShapeDtypeStruct((rows, 2), _f32),
    )(hs, hd, efp, scale, shift, W1p, b1, W2, b2, W3, b3, W4, b4, W5, b5,
      W6, b6)


def kernel(x, edge_index, pos_src, pos_dst, edge_feat, labels, emb,
           W_self1, W_neigh1, b1, W_self2, W_neigh2, b2,
           bn_gamma, bn_beta,
           W_d1, b_d1, W_d2, b_d2, W_d3, b_d3, W_d4, b_d4, W_d5, b_d5,
           W_d6, b_d6):
    x = x.astype(_i32)
    src = edge_index[0].astype(_i32)
    dst = edge_index[1].astype(_i32)
    pos_src = pos_src.astype(_i32)
    pos_dst = pos_dst.astype(_i32)

    x2d = jnp.concatenate([x, jnp.zeros((NP - N,), _i32)]) \
        .reshape(NW, NP // (NW * 64), 64)
    srcp = jnp.concatenate([src, jnp.zeros((EPAD - E,), _i32)]) \
        .reshape(NS, EPAD // (NS * 128), 128)
    dstp = jnp.concatenate([dst, jnp.full((EPAD - E,), N, _i32)]) \
        .reshape(NS, EPAD // (NS * 128), 128)
    pos_s_pad = jnp.concatenate([pos_src, jnp.full((EPP - EP,), N, _i32)])
    pos_d_pad = jnp.concatenate([pos_dst, jnp.full((EPP - EP,), N, _i32)])
    psp = pos_s_pad.reshape(NW, EPP // (NW * 128), 128)
    pdp = pos_d_pad.reshape(NW, EPP // (NW * 128), 128)
    efp = jnp.pad(edge_feat, ((0, EPP - EP), (0, HH - EF)))
    zfeat = jnp.zeros((NP // NS, 128), _f32)
    ones128 = jnp.ones((128, 128), _f32)
    dstw = jnp.concatenate([dst, jnp.full((EPAD - E,), N, _i32)]) \
        .reshape(NW, EPAD // (NW * 128), 128)
    b1r = b1.reshape(1, H)
    b2r = b2.reshape(1, H)

    h0 = _emb_gather(x2d, emb)

    neigh1 = _segsum(h0.reshape(2 * NP, 128), srcp, dstp, zfeat)
    deg = _deg(dstw, ones128, zfeat)
    h1 = _sage_combine(h0, neigh1, deg, W_self1, W_neigh1, b1r,
                       mask_pad=False)

    neigh2 = _segsum(h1.reshape(2 * NP, 128), srcp, dstp, zfeat)
    h2 = _sage_combine(h1, neigh2, deg, W_self2, W_neigh2, b2r,
                       mask_pad=True)

    cnt_s = _deg(psp, ones128, zfeat)
    cnt_d = _deg(pdp, ones128, zfeat)


    efs = _ef_stats(efp)
    h2s = _h2_stats(h2, cnt_s, cnt_d)
    sums = jnp.concatenate([h2s[0], h2s[2], efs[0]])
    sumsqs = jnp.concatenate([h2s[1], h2s[3], efs[1]])
    mean = sums / EP
    var = sumsqs / EP - mean * mean
    gamma = jnp.pad(bn_gamma, (0, AUGP - (2 * H + EF)))
    beta = jnp.pad(bn_beta, (0, AUGP - (2 * H + EF)))
    inv_std = 1.0 / jnp.sqrt(var + 1e-5)
    scale = (gamma * inv_std).reshape(1, AUGP)
    shift = (beta - mean * gamma * inv_std).reshape(1, AUGP)

    W1p = jnp.pad(W_d1, ((0, AUGP - (2 * H + EF)), (0, 0)))
    mlp_w = (W1p, b_d1.reshape(1, MLPD),
             W_d2, b_d2.reshape(1, MLPD),
             W_d3, b_d3.reshape(1, MLPD),
             W_d4, b_d4.reshape(1, MLPD),
             W_d5, b_d5.reshape(1, MLPD),
             W_d6, b_d6.reshape(1, 2))
    piece_chunks = (5, 5, 5, 5, 5)
    probs = []
    r0 = 0
    for pc in piece_chunks:
        rows = NW * pc * 128
        psp_i = lax.dynamic_slice_in_dim(pos_s_pad, r0, rows) \
            .reshape(NW, pc, 128)
        pdp_i = lax.dynamic_slice_in_dim(pos_d_pad, r0, rows) \
            .reshape(NW, pc, 128)
        hs_i, hd_i = _pair_gather(h2, psp_i, pdp_i)
        probs.append(_mlp(hs_i, hd_i, efp, r0, scale, shift, *mlp_w))
        r0 += rows
    probs = jnp.concatenate(probs, axis=0)
    return (probs[:EP], labels.reshape(-1, 1))


# speedup vs baseline: 1.0228x; 1.0228x over previous
import functools

import jax
import jax.numpy as jnp
from jax import lax
from jax.experimental import pallas as pl
from jax.experimental.pallas import tpu as pltpu
from jax.experimental.pallas import tpu_sc as plsc

N = 10000
NP = 10240
E = 160000
EPAD = 163840
EP = 100000
EPP = 102400
H = 256
HH = 128
MLPD = 512
EF = 75
AUGP = 640

NC = 2
NS = 16
NW = NC * NS

_f32 = jnp.float32
_i32 = jnp.int32


def _sc_mesh():
    return plsc.VectorSubcoreMesh(core_axis_name="c", subcore_axis_name="s")


def _emb_gather(x2d, emb):
    nrows = NP // 64
    per_w = nrows // NW

    def body(x_ref, emb_ref, out_ref, idx_v, rows_v,
             sem_g0, sem_g1, sem_w0, sem_w1):
        c = lax.axis_index("c")
        s = lax.axis_index("s")
        w = s * NC + c
        sem_g = (sem_g0, sem_g1)
        sem_w = (sem_w0, sem_w1)
        pltpu.sync_copy(x_ref.at[w], idx_v)
        gathers = [None] * per_w
        writes = [None] * per_w
        for j in range(per_w):
            b = j % 2
            if j >= 2:
                writes[j - 2].wait()
            gathers[j] = pltpu.async_copy(
                emb_ref.at[idx_v.at[j]], rows_v.at[b], sem_g[b])
            if j >= 1:
                p = j - 1
                gathers[p].wait()
                base = pl.multiple_of((w * per_w + p) * 64, 64)
                writes[p] = pltpu.async_copy(
                    rows_v.at[p % 2], out_ref.at[pl.ds(base, 64)],
                    sem_w[p % 2])
        p = per_w - 1
        gathers[p].wait()
        base = pl.multiple_of((w * per_w + p) * 64, 64)
        writes[p] = pltpu.async_copy(
            rows_v.at[p % 2], out_ref.at[pl.ds(base, 64)], sem_w[p % 2])
        writes[per_w - 2].wait()
        writes[per_w - 1].wait()

    k = pl.kernel(
        body,
        out_type=jax.ShapeDtypeStruct((NP, H), _f32),
        mesh=_sc_mesh(),
        scratch_types=[
            pltpu.VMEM((per_w, 64), _i32),
            pltpu.VMEM((2, 64, H), _f32),
            pltpu.SemaphoreType.DMA,
            pltpu.SemaphoreType.DMA,
            pltpu.SemaphoreType.DMA,
            pltpu.SemaphoreType.DMA,
        ],
    )
    return k(x2d, emb)


def _segsum(hf, srcp, dstp, zfeat):
    rows_per_tile = NP // NS
    chunks = EPAD // (NS * 128)
    SUP = 8

    def body(h_ref, src_ref, dst_ref, zf_ref, neigh_ref,
             src_v, dst_v, gidx_v, rows_v, acc,
             sem_g0, sem_g1, sem_s0, sem_s1):
        c = lax.axis_index("c")
        s = lax.axis_index("s")
        stripe = pl.ds(pl.multiple_of(s * rows_per_tile, rows_per_tile),
                       rows_per_tile)
        sem_g = (sem_g0, sem_g1)
        sem_s = (sem_s0, sem_s1)
        pltpu.sync_copy(zf_ref, acc.at[stripe])
        plsc.subcore_barrier()

        gathers = [None] * chunks
        scats = [None] * chunks
        for t in range(chunks):
            b = t % 2
            sup, j = divmod(t, SUP)
            sb = sup % 2
            if j == 0:
                koff = pl.multiple_of(sup * SUP, SUP)
                pltpu.sync_copy(src_ref.at[s, pl.ds(koff, SUP)],
                                src_v.at[sb])
                pltpu.sync_copy(dst_ref.at[s, pl.ds(koff, SUP)],
                                dst_v.at[sb])
            if t >= 2:
                scats[t - 2].wait()
            for i in range(8):
                sl = pl.ds(i * 16, 16)
                gidx_v[b, sl] = src_v[sb, j, sl] * 2 + c
            gathers[t] = pltpu.async_copy(
                h_ref.at[gidx_v.at[b]], rows_v.at[b], sem_g[b])
            if t >= 1:
                p = t - 1
                gathers[p].wait()
                psup, pj = divmod(p, SUP)
                scats[p] = pltpu.async_copy(
                    rows_v.at[p % 2], acc.at[dst_v.at[psup % 2, pj]],
                    sem_s[p % 2], add=True)
        p = chunks - 1
        gathers[p].wait()
        psup, pj = divmod(p, SUP)
        scats[p] = pltpu.async_copy(
            rows_v.at[p % 2], acc.at[dst_v.at[psup % 2, pj]],
            sem_s[p % 2], add=True)
        scats[chunks - 2].wait()
        scats[chunks - 1].wait()
        plsc.subcore_barrier()
        pltpu.sync_copy(acc.at[stripe], neigh_ref.at[c, stripe])

    k = pl.kernel(
        body,
        out_type=jax.ShapeDtypeStruct((2, NP, 128), _f32),
        mesh=_sc_mesh(),
        scratch_types=[
            pltpu.VMEM((2, SUP, 128), _i32),
            pltpu.VMEM((2, SUP, 128), _i32),
            pltpu.VMEM((2, 128), _i32),
            pltpu.VMEM((2, 128, 128), _f32),
            pltpu.VMEM_SHARED((NP, 128), _f32),
            pltpu.SemaphoreType.DMA,
            pltpu.SemaphoreType.DMA,
            pltpu.SemaphoreType.DMA,
            pltpu.SemaphoreType.DMA,
        ],
    )
    return k(hf, srcp, dstp, zfeat)


def _deg(dstw, ones_hbm, zfeat):
    rows_per_tile = NP // NS
    chunks = dstw.shape[1]

    def body(dst_ref, ones_ref, zf_ref, deg_ref, dst_v, ones_v, acc):
        c = lax.axis_index("c")
        s = lax.axis_index("s")
        w = s * NC + c
        stripe = pl.ds(pl.multiple_of(s * rows_per_tile, rows_per_tile),
                       rows_per_tile)
        pltpu.sync_copy(zf_ref, acc.at[stripe])
        pltpu.sync_copy(ones_ref, ones_v)
        pltpu.sync_copy(dst_ref.at[w], dst_v)
        plsc.subcore_barrier()

        def chunk(j, _):
            pltpu.sync_copy(ones_v, acc.at[dst_v.at[j]], add=True)
            return 0

        lax.fori_loop(0, chunks, chunk, 0)
        plsc.subcore_barrier()
        pltpu.sync_copy(acc.at[stripe], deg_ref.at[c, stripe])

    k = pl.kernel(
        body,
        out_type=jax.ShapeDtypeStruct((2, NP, 128), _f32),
        mesh=_sc_mesh(),
        scratch_types=[
            pltpu.VMEM((chunks, 128), _i32),
            pltpu.VMEM((128, 128), _f32),
            pltpu.VMEM_SHARED((NP, 128), _f32),
        ],
    )
    return k(dstw, ones_hbm, zfeat)


def _pair_gather(h2, psp, pdp):
    chunks = psp.shape[1]

    def body(h_ref, ps_ref, pd_ref, hs_ref, hd_ref,
             is_v, id_v, rows_v, sem_g0, sem_g1, sem_w0, sem_w1):
        c = lax.axis_index("c")
        s = lax.axis_index("s")
        w = s * NC + c
        sem_g = (sem_g0, sem_g1)
        sem_w = (sem_w0, sem_w1)
        pltpu.sync_copy(ps_ref.at[w], is_v)
        pltpu.sync_copy(pd_ref.at[w], id_v)

        n = 2 * chunks
        gathers = [None] * n
        writes = [None] * n

        def issue_write(u):
            j, which = divmod(u, 2)
            base = pl.multiple_of((w * chunks + j) * 128, 128)
            out = hs_ref if which == 0 else hd_ref
            writes[u] = pltpu.async_copy(
                rows_v.at[u % 2], out.at[pl.ds(base, 128)], sem_w[u % 2])

        for u in range(n):
            b = u % 2
            j, which = divmod(u, 2)
            if u >= 2:
                writes[u - 2].wait()
            idx = is_v if which == 0 else id_v
            gathers[u] = pltpu.async_copy(
                h_ref.at[idx.at[j]], rows_v.at[b], sem_g[b])
            if u >= 1:
                gathers[u - 1].wait()
                issue_write(u - 1)
        gathers[n - 1].wait()
        issue_write(n - 1)
        writes[n - 2].wait()
        writes[n - 1].wait()

    rows = NW * chunks * 128
    k = pl.kernel(
        body,
        out_type=(jax.ShapeDtypeStruct((rows, H), _f32),
                  jax.ShapeDtypeStruct((rows, H), _f32)),
        mesh=_sc_mesh(),
        scratch_types=[
            pltpu.VMEM((chunks, 128), _i32),
            pltpu.VMEM((chunks, 128), _i32),
            pltpu.VMEM((2, 128, H), _f32),
            pltpu.SemaphoreType.DMA,
            pltpu.SemaphoreType.DMA,
            pltpu.SemaphoreType.DMA,
            pltpu.SemaphoreType.DMA,
        ],
    )
    return k(h2, psp, pdp)


def _sage_combine(h, neigh, deg, Ws, Wn, b, mask_pad, out_bf16=False):
    BS = 1024
    grid = NP // BS
    odt = jnp.bfloat16 if out_bf16 else _f32

    def body(h_ref, n_ref, d_ref, ws_ref, wn_ref, b_ref, out_ref):
        g = pl.program_id(0)
        deg_col = d_ref[0][:, 0:1] + d_ref[1][:, 0:1]
        r = 1.0 / jnp.clip(deg_col, 1.0, None)
        nm = sum(
            jax.lax.dot_general(n_ref[q], wn_ref[q * 128:(q + 1) * 128, :],
                                (((1,), (0,)), ((), ())),
                                preferred_element_type=_f32)
            for q in range(2))
        out = (jnp.dot(h_ref[...], ws_ref[...], preferred_element_type=_f32)
               + nm * r + b_ref[...])
        if mask_pad:
            rows = g * BS + lax.broadcasted_iota(_i32, (BS, H), 0)
            out = jnp.where(rows < N, out, 0.0)
        out_ref[...] = out.astype(odt)

    return pl.pallas_call(
        body,
        grid=(grid,),
        in_specs=[
            pl.BlockSpec((BS, H), lambda g: (g, 0)),
            pl.BlockSpec((2, BS, 128), lambda g: (0, g, 0)),
            pl.BlockSpec((2, BS, 128), lambda g: (0, g, 0)),
            pl.BlockSpec((H, H), lambda g: (0, 0)),
            pl.BlockSpec((H, H), lambda g: (0, 0)),
            pl.BlockSpec((1, H), lambda g: (0, 0)),
        ],
        out_specs=pl.BlockSpec((BS, H), lambda g: (g, 0)),
        out_shape=jax.ShapeDtypeStruct((NP, H), odt),
    )(h, neigh, deg, Ws, Wn, b)


def _ef_stats(efp):
    BS = 1600
    grid = EPP // BS

    def body(ef_ref, out_ref):
        g = pl.program_id(0)
        s1 = jnp.sum(ef_ref[...], axis=0)[None, :]
        s2 = jnp.sum(ef_ref[...] * ef_ref[...], axis=0)[None, :]
        upd = jnp.concatenate([s1, s2, jnp.zeros((6, HH), _f32)], axis=0)

        @pl.when(g == 0)
        def _():
            out_ref[...] = upd

        @pl.when(g > 0)
        def _():
            out_ref[...] = out_ref[...] + upd

    return pl.pallas_call(
        body,
        grid=(grid,),
        in_specs=[pl.BlockSpec((BS, HH), lambda g: (g, 0))],
        out_specs=pl.BlockSpec((8, HH), lambda g: (0, 0)),
        out_shape=jax.ShapeDtypeStruct((8, HH), _f32),
        compiler_params=pltpu.CompilerParams(
            dimension_semantics=("arbitrary",)),
    )(efp)


def _h2_stats(h2, cs, cd):
    BS = 1024
    grid = NP // BS

    def body(h_ref, cs_ref, cd_ref, out_ref):
        g = pl.program_id(0)
        h = h_ref[...]
        hsq = h * h
        cs_col = cs_ref[0][:, 0:1] + cs_ref[1][:, 0:1]
        cd_col = cd_ref[0][:, 0:1] + cd_ref[1][:, 0:1]
        upd = jnp.concatenate([
            jnp.sum(h * cs_col, axis=0)[None, :],
            jnp.sum(hsq * cs_col, axis=0)[None, :],
            jnp.sum(h * cd_col, axis=0)[None, :],
            jnp.sum(hsq * cd_col, axis=0)[None, :],
            jnp.zeros((4, H), _f32),
        ], axis=0)

        @pl.when(g == 0)
        def _():
            out_ref[...] = upd

        @pl.when(g > 0)
        def _():
            out_ref[...] = out_ref[...] + upd

    return pl.pallas_call(
        body,
        grid=(grid,),
        in_specs=[
            pl.BlockSpec((BS, H), lambda g: (g, 0)),
            pl.BlockSpec((2, BS, 128), lambda g: (0, g, 0)),
            pl.BlockSpec((2, BS, 128), lambda g: (0, g, 0)),
        ],
        out_specs=pl.BlockSpec((8, H), lambda g: (0, 0)),
        out_shape=jax.ShapeDtypeStruct((8, H), _f32),
        compiler_params=pltpu.CompilerParams(
            dimension_semantics=("arbitrary",)),
    )(h2, cs, cd)


def _mlp(hs, hd, efp, ef_row0, scale, shift, W1p, b1, W2, b2, W3, b3, W4,
         b4, W5, b5, W6, b6):
    BS = 1024
    rows = hs.shape[0]
    grid = rows // BS
    ef_blk0 = ef_row0 // BS

    def body(hs_ref, hd_ref, ef_ref, sc_ref, sh_ref, w1_ref, b1_ref,
             w2_ref, b2_ref, w3_ref, b3_ref, w4_ref, b4_ref, w5_ref, b5_ref,
             w6_ref, b6_ref, out_ref):
        x = jnp.concatenate([hs_ref[...], hd_ref[...], ef_ref[...]], axis=1)
        a = x * sc_ref[...] + sh_ref[...]
        for w_ref, bb_ref in ((w1_ref, b1_ref), (w2_ref, b2_ref),
                              (w3_ref, b3_ref), (w4_ref, b4_ref),
                              (w5_ref, b5_ref)):
            a = jnp.maximum(
                jnp.dot(a, w_ref[...], preferred_element_type=_f32)
                + bb_ref[...], 0.0)
        logits = jnp.dot(a, w6_ref[...], preferred_element_type=_f32) \
            + b6_ref[...]
        l0 = logits[:, 0:1]
        l1 = logits[:, 1:2]
        m = jnp.maximum(l0, l1)
        e0 = jnp.exp(l0 - m)
        e1 = jnp.exp(l1 - m)
        inv = 1.0 / (e0 + e1)
        out_ref[...] = jnp.concatenate([e0 * inv, e1 * inv], axis=1)

    def wspec(shape):
        return pl.BlockSpec(shape, lambda g, _n=len(shape): (0,) * _n)

    return pl.pallas_call(
        body,
        grid=(grid,),
        in_specs=[
            pl.BlockSpec((BS, H), lambda g: (g, 0)),
            pl.BlockSpec((BS, H), lambda g: (g, 0)),
            pl.BlockSpec((BS, HH), lambda g: (ef_blk0 + g, 0)),
            wspec((1, AUGP)), wspec((1, AUGP)),
            wspec((AUGP, MLPD)), wspec((1, MLPD)),
            wspec((MLPD, MLPD)), wspec((1, MLPD)),
            wspec((MLPD, MLPD)), wspec((1, MLPD)),
            wspec((MLPD, MLPD)), wspec((1, MLPD)),
            wspec((MLPD, MLPD)), wspec((1, MLPD)),
            wspec((MLPD, 2)), wspec((1, 2)),
        ],
        out_specs=pl.BlockSpec((BS, 2), lambda g: (g, 0)),
        out_shape=jax.ShapeDtypeStruct((rows, 2), _f32),
    )(hs, hd, efp, scale, shift, W1p, b1, W2, b2, W3, b3, W4, b4, W5, b5,
      W6, b6)


def kernel(x, edge_index, pos_src, pos_dst, edge_feat, labels, emb,
           W_self1, W_neigh1, b1, W_self2, W_neigh2, b2,
           bn_gamma, bn_beta,
           W_d1, b_d1, W_d2, b_d2, W_d3, b_d3, W_d4, b_d4, W_d5, b_d5,
           W_d6, b_d6):
    x = x.astype(_i32)
    src = edge_index[0].astype(_i32)
    dst = edge_index[1].astype(_i32)
    pos_src = pos_src.astype(_i32)
    pos_dst = pos_dst.astype(_i32)

    x2d = jnp.concatenate([x, jnp.zeros((NP - N,), _i32)]) \
        .reshape(NW, NP // (NW * 64), 64)
    srcp = jnp.concatenate([src, jnp.zeros((EPAD - E,), _i32)]) \
        .reshape(NS, EPAD // (NS * 128), 128)
    dstp = jnp.concatenate([dst, jnp.full((EPAD - E,), N, _i32)]) \
        .reshape(NS, EPAD // (NS * 128), 128)
    pos_s_pad = jnp.concatenate([pos_src, jnp.full((EPP - EP,), N, _i32)])
    pos_d_pad = jnp.concatenate([pos_dst, jnp.full((EPP - EP,), N, _i32)])
    psp = pos_s_pad.reshape(NW, EPP // (NW * 128), 128)
    pdp = pos_d_pad.reshape(NW, EPP // (NW * 128), 128)
    efp = jnp.pad(edge_feat, ((0, EPP - EP), (0, HH - EF)))
    zfeat = jnp.zeros((NP // NS, 128), _f32)
    ones128 = jnp.ones((128, 128), _f32)
    dstw = jnp.concatenate([dst, jnp.full((EPAD - E,), N, _i32)]) \
        .reshape(NW, EPAD // (NW * 128), 128)
    b1r = b1.reshape(1, H)
    b2r = b2.reshape(1, H)

    h0 = _emb_gather(x2d, emb)

    neigh1 = _segsum(h0.reshape(2 * NP, 128), srcp, dstp, zfeat)
    deg = _deg(dstw, ones128, zfeat)
    h1 = _sage_combine(h0, neigh1, deg, W_self1, W_neigh1, b1r,
                       mask_pad=False)

    neigh2 = _segsum(h1.reshape(2 * NP, 128), srcp, dstp, zfeat)
    h2 = _sage_combine(h1, neigh2, deg, W_self2, W_neigh2, b2r,
                       mask_pad=True)

    cnt_s = _deg(psp, ones128, zfeat)
    cnt_d = _deg(pdp, ones128, zfeat)


    efs = _ef_stats(efp)
    h2s = _h2_stats(h2, cnt_s, cnt_d)
    sums = jnp.concatenate([h2s[0], h2s[2], efs[0]])
    sumsqs = jnp.concatenate([h2s[1], h2s[3], efs[1]])
    mean = sums / EP
    var = sumsqs / EP - mean * mean
    gamma = jnp.pad(bn_gamma, (0, AUGP - (2 * H + EF)))
    beta = jnp.pad(bn_beta, (0, AUGP - (2 * H + EF)))
    inv_std = 1.0 / jnp.sqrt(var + 1e-5)
    scale = (gamma * inv_std).reshape(1, AUGP)
    shift = (beta - mean * gamma * inv_std).reshape(1, AUGP)

    W1p = jnp.pad(W_d1, ((0, AUGP - (2 * H + EF)), (0, 0)))
    mlp_w = (W1p, b_d1.reshape(1, MLPD),
             W_d2, b_d2.reshape(1, MLPD),
             W_d3, b_d3.reshape(1, MLPD),
             W_d4, b_d4.reshape(1, MLPD),
             W_d5, b_d5.reshape(1, MLPD),
             W_d6, b_d6.reshape(1, 2))
    piece_chunks = (9, 8, 8)
    probs = []
    r0 = 0
    for pc in piece_chunks:
        rows = NW * pc * 128
        psp_i = lax.dynamic_slice_in_dim(pos_s_pad, r0, rows) \
            .reshape(NW, pc, 128)
        pdp_i = lax.dynamic_slice_in_dim(pos_d_pad, r0, rows) \
            .reshape(NW, pc, 128)
        hs_i, hd_i = _pair_gather(h2, psp_i, pdp_i)
        probs.append(_mlp(hs_i, hd_i, efp, r0, scale, shift, *mlp_w))
        r0 += rows
    probs = jnp.concatenate(probs, axis=0)
    return (probs[:EP], labels.reshape(-1, 1))
